# Initial kernel scaffold; baseline (speedup 1.0000x reference)
import functools

import jax
import jax.numpy as jnp
from jax import lax
from jax.experimental import pallas as pl
from jax.experimental.pallas import tpu as pltpu
from jax.experimental.pallas import tpu_sc as plsc

_EMB = 128
_NC, _NS = 2, 16
_NW = _NC * _NS
_C = 128

_BT, _LT = 1024, 200
_LI = 20
_NT = _BT * _LT
_NI = _BT * _LI
_TCHUNKS = _NT // (_NW * _C)
_ICHUNKS = _NI // (_NW * _C)
_IMG_IN = 2048


def _sc_body(tok, post, t0, t1, t2, tp, i0, i1, i2,
             text_out, img_out,
             v_t0, v_t1, v_t2, v_tp, v_i0, v_i1, v_i2,
             b0, b1, b2, b3, sem):
    wid = lax.axis_index("s") * _NC + lax.axis_index("c")

    pltpu.sync_copy(t0.at[pl.ds(wid * _TCHUNKS, _TCHUNKS)], v_t0)
    pltpu.sync_copy(t1.at[pl.ds(wid * _TCHUNKS, _TCHUNKS)], v_t1)
    pltpu.sync_copy(t2.at[pl.ds(wid * _TCHUNKS, _TCHUNKS)], v_t2)
    pltpu.sync_copy(tp.at[pl.ds(wid * _TCHUNKS, _TCHUNKS)], v_tp)
    pltpu.sync_copy(i0.at[pl.ds(wid * _ICHUNKS, _ICHUNKS)], v_i0)
    pltpu.sync_copy(i1.at[pl.ds(wid * _ICHUNKS, _ICHUNKS)], v_i1)
    pltpu.sync_copy(i2.at[pl.ds(wid * _ICHUNKS, _ICHUNKS)], v_i2)

    def _sum_rows(nbuf):
        def row(r, _):
            for j in range(_EMB // 16):
                s = pl.ds(j * 16, 16)
                acc = b0[r, s] + b1[r, s] + b2[r, s]
                if nbuf == 4:
                    acc = acc + b3[r, s]
                b0[r, s] = acc
            return 0
        lax.fori_loop(0, _C, row, 0)

    def text_chunk(g, _):
        c0 = pltpu.async_copy(tok.at[v_t0.at[g]], b0, sem)
        c1 = pltpu.async_copy(tok.at[v_t1.at[g]], b1, sem)
        c2 = pltpu.async_copy(tok.at[v_t2.at[g]], b2, sem)
        c3 = pltpu.async_copy(post.at[v_tp.at[g]], b3, sem)
        c0.wait(); c1.wait(); c2.wait(); c3.wait()
        _sum_rows(4)
        base = wid * (_TCHUNKS * _C) + g * _C
        pltpu.sync_copy(b0, text_out.at[pl.ds(base, _C)])
        return 0

    lax.fori_loop(0, _TCHUNKS, text_chunk, 0)

    def img_chunk(g, _):
        c0 = pltpu.async_copy(tok.at[v_i0.at[g]], b0, sem)
        c1 = pltpu.async_copy(tok.at[v_i1.at[g]], b1, sem)
        c2 = pltpu.async_copy(tok.at[v_i2.at[g]], b2, sem)
        c0.wait(); c1.wait(); c2.wait()
        _sum_rows(3)
        base = wid * (_ICHUNKS * _C) + g * _C
        pltpu.sync_copy(b0, img_out.at[pl.ds(base, _C)])
        return 0

    lax.fori_loop(0, _ICHUNKS, img_chunk, 0)


_sc_gather = functools.partial(
    pl.kernel,
    out_type=(
        jax.ShapeDtypeStruct((_NT, _EMB), jnp.float32),
        jax.ShapeDtypeStruct((_NI, _EMB), jnp.float32),
    ),
    mesh=plsc.VectorSubcoreMesh(core_axis_name="c", subcore_axis_name="s"),
    scratch_types=[
        pltpu.VMEM((_TCHUNKS, _C), jnp.int32),
        pltpu.VMEM((_TCHUNKS, _C), jnp.int32),
        pltpu.VMEM((_TCHUNKS, _C), jnp.int32),
        pltpu.VMEM((_TCHUNKS, _C), jnp.int32),
        pltpu.VMEM((_ICHUNKS, _C), jnp.int32),
        pltpu.VMEM((_ICHUNKS, _C), jnp.int32),
        pltpu.VMEM((_ICHUNKS, _C), jnp.int32),
        pltpu.VMEM((_C, _EMB), jnp.float32),
        pltpu.VMEM((_C, _EMB), jnp.float32),
        pltpu.VMEM((_C, _EMB), jnp.float32),
        pltpu.VMEM((_C, _EMB), jnp.float32),
        pltpu.SemaphoreType.DMA,
    ],
)(_sc_body)


def _mm_body(x_ref, w_ref, g_ref, o_ref):
    o_ref[...] = (
        jnp.dot(x_ref[...], w_ref[...], preferred_element_type=jnp.float32)
        + g_ref[...]
    )


def _img_project(x, w_t, gsum):
    m_blk = 1024
    return pl.pallas_call(
        _mm_body,
        grid=(_NI // m_blk,),
        in_specs=[
            pl.BlockSpec((m_blk, _IMG_IN), lambda i: (i, 0)),
            pl.BlockSpec((_IMG_IN, _EMB), lambda i: (0, 0)),
            pl.BlockSpec((m_blk, _EMB), lambda i: (i, 0)),
        ],
        out_specs=pl.BlockSpec((m_blk, _EMB), lambda i: (i, 0)),
        out_shape=jax.ShapeDtypeStruct((_NI, _EMB), jnp.float32),
    )(x, w_t, gsum)


def kernel(src_input, src_pos, src_turn, src_speaker, image_input,
           image_pos, image_turn, image_speaker, tok_table, pos_table, W_img):
    i32 = jnp.int32
    tc = _NT // _C
    ic = _NI // _C
    t0 = src_input.reshape(tc, _C).astype(i32)
    t1 = src_turn.reshape(tc, _C).astype(i32)
    t2 = src_speaker.reshape(tc, _C).astype(i32)
    tp = src_pos.reshape(tc, _C).astype(i32)
    i0 = image_turn.reshape(ic, _C).astype(i32)
    i1 = image_speaker.reshape(ic, _C).astype(i32)
    i2 = image_pos.reshape(ic, _C).astype(i32)

    text_flat, img_gather = _sc_gather(
        tok_table, pos_table, t0, t1, t2, tp, i0, i1, i2)

    x = image_input.reshape(_NI, _IMG_IN)
    img_flat = _img_project(x, W_img.T, img_gather)

    return (text_flat.reshape(_BT, _LT, _EMB),
            img_flat.reshape(_BT, _LI, _EMB))


# speedup vs baseline: 4.5743x; 4.5743x over previous
import functools

import jax
import jax.numpy as jnp
from jax import lax
from jax.experimental import pallas as pl
from jax.experimental.pallas import tpu as pltpu
from jax.experimental.pallas import tpu_sc as plsc

_EMB = 128
_NC, _NS = 2, 16
_NW = _NC * _NS
_C = 128

_BT, _LT = 1024, 200
_LI = 20
_NT = _BT * _LT
_NI = _BT * _LI
_TCHUNKS = _NT // (_NW * _C)
_ICHUNKS = _NI // (_NW * _C)
_IMG_IN = 2048


def _sc_body(tok, post, t0, t1, t2, tp, i0, i1, i2,
             text_out, img_out,
             v_t0, v_t1, v_t2, v_tp, v_i0, v_i1, v_i2,
             b0, b1, b2, b3, sem):
    wid = lax.axis_index("s") * _NC + lax.axis_index("c")

    pltpu.sync_copy(t0.at[wid], v_t0)
    pltpu.sync_copy(t1.at[wid], v_t1)
    pltpu.sync_copy(t2.at[wid], v_t2)
    pltpu.sync_copy(tp.at[wid], v_tp)
    pltpu.sync_copy(i0.at[wid], v_i0)
    pltpu.sync_copy(i1.at[wid], v_i1)
    pltpu.sync_copy(i2.at[wid], v_i2)

    def _sum_rows(nbuf):
        def row(r, _):
            for j in range(_EMB // 16):
                s = pl.ds(j * 16, 16)
                acc = b0[r, s] + b1[r, s] + b2[r, s]
                if nbuf == 4:
                    acc = acc + b3[r, s]
                b0[r, s] = acc
            return 0
        lax.fori_loop(0, _C, row, 0)

    def text_chunk(g, _):
        c0 = pltpu.async_copy(tok.at[v_t0.at[g]], b0, sem)
        c1 = pltpu.async_copy(tok.at[v_t1.at[g]], b1, sem)
        c2 = pltpu.async_copy(tok.at[v_t2.at[g]], b2, sem)
        c3 = pltpu.async_copy(post.at[v_tp.at[g]], b3, sem)
        c0.wait(); c1.wait(); c2.wait(); c3.wait()
        _sum_rows(4)
        base = wid * (_TCHUNKS * _C) + g * _C
        pltpu.sync_copy(b0, text_out.at[pl.ds(base, _C)])
        return 0

    lax.fori_loop(0, _TCHUNKS, text_chunk, 0)

    def img_chunk(g, _):
        c0 = pltpu.async_copy(tok.at[v_i0.at[g]], b0, sem)
        c1 = pltpu.async_copy(tok.at[v_i1.at[g]], b1, sem)
        c2 = pltpu.async_copy(tok.at[v_i2.at[g]], b2, sem)
        c0.wait(); c1.wait(); c2.wait()
        _sum_rows(3)
        base = wid * (_ICHUNKS * _C) + g * _C
        pltpu.sync_copy(b0, img_out.at[pl.ds(base, _C)])
        return 0

    lax.fori_loop(0, _ICHUNKS, img_chunk, 0)


_sc_gather = functools.partial(
    pl.kernel,
    out_type=(
        jax.ShapeDtypeStruct((_NT, _EMB), jnp.float32),
        jax.ShapeDtypeStruct((_NI, _EMB), jnp.float32),
    ),
    mesh=plsc.VectorSubcoreMesh(core_axis_name="c", subcore_axis_name="s"),
    scratch_types=[
        pltpu.VMEM((_TCHUNKS, _C), jnp.int32),
        pltpu.VMEM((_TCHUNKS, _C), jnp.int32),
        pltpu.VMEM((_TCHUNKS, _C), jnp.int32),
        pltpu.VMEM((_TCHUNKS, _C), jnp.int32),
        pltpu.VMEM((_ICHUNKS, _C), jnp.int32),
        pltpu.VMEM((_ICHUNKS, _C), jnp.int32),
        pltpu.VMEM((_ICHUNKS, _C), jnp.int32),
        pltpu.VMEM((_C, _EMB), jnp.float32),
        pltpu.VMEM((_C, _EMB), jnp.float32),
        pltpu.VMEM((_C, _EMB), jnp.float32),
        pltpu.VMEM((_C, _EMB), jnp.float32),
        pltpu.SemaphoreType.DMA,
    ],
)(_sc_body)


def _mm_body(x_ref, w_ref, g_ref, o_ref):
    o_ref[...] = (
        jnp.dot(x_ref[...], w_ref[...], preferred_element_type=jnp.float32)
        + g_ref[...]
    )


def _img_project(x, w_t, gsum):
    m_blk = 1024
    return pl.pallas_call(
        _mm_body,
        grid=(_NI // m_blk,),
        in_specs=[
            pl.BlockSpec((m_blk, _IMG_IN), lambda i: (i, 0)),
            pl.BlockSpec((_IMG_IN, _EMB), lambda i: (0, 0)),
            pl.BlockSpec((m_blk, _EMB), lambda i: (i, 0)),
        ],
        out_specs=pl.BlockSpec((m_blk, _EMB), lambda i: (i, 0)),
        out_shape=jax.ShapeDtypeStruct((_NI, _EMB), jnp.float32),
    )(x, w_t, gsum)


def kernel(src_input, src_pos, src_turn, src_speaker, image_input,
           image_pos, image_turn, image_speaker, tok_table, pos_table, W_img):
    i32 = jnp.int32
    t0 = src_input.reshape(_NW, _TCHUNKS, _C).astype(i32)
    t1 = src_turn.reshape(_NW, _TCHUNKS, _C).astype(i32)
    t2 = src_speaker.reshape(_NW, _TCHUNKS, _C).astype(i32)
    tp = src_pos.reshape(_NW, _TCHUNKS, _C).astype(i32)
    i0 = image_turn.reshape(_NW, _ICHUNKS, _C).astype(i32)
    i1 = image_speaker.reshape(_NW, _ICHUNKS, _C).astype(i32)
    i2 = image_pos.reshape(_NW, _ICHUNKS, _C).astype(i32)

    text_flat, img_gather = _sc_gather(
        tok_table, pos_table, t0, t1, t2, tp, i0, i1, i2)

    x = image_input.reshape(_NI, _IMG_IN)
    img_flat = _img_project(x, W_img.T, img_gather)

    return (text_flat.reshape(_BT, _LT, _EMB),
            img_flat.reshape(_BT, _LI, _EMB))


# speedup vs baseline: 7.8493x; 1.7160x over previous
import functools

import jax
import jax.numpy as jnp
from jax import lax
from jax.experimental import pallas as pl
from jax.experimental.pallas import tpu as pltpu
from jax.experimental.pallas import tpu_sc as plsc

_EMB = 128
_NC, _NS = 2, 16
_NW = _NC * _NS
_C = 128

_BT, _LT = 1024, 200
_LI = 20
_NT = _BT * _LT
_NI = _BT * _LI
_TCHUNKS = _NT // (_NW * _C)
_ICHUNKS = _NI // (_NW * _C)
_IMG_IN = 2048

_MESH = plsc.VectorSubcoreMesh(core_axis_name="c", subcore_axis_name="s")
_PARAMS = pltpu.CompilerParams(use_tc_tiling_on_sc=True)


def _sc_text_body(tok, post, t0, t1, t2, tp, td, text_out,
                  v_t0, v_t1, v_t2, v_tp, v_td,
                  bA, bB, gsA, gsB, ssA, ssB):
    wid = lax.axis_index("s") * _NC + lax.axis_index("c")

    pltpu.sync_copy(t0.at[wid], v_t0)
    pltpu.sync_copy(t1.at[wid], v_t1)
    pltpu.sync_copy(t2.at[wid], v_t2)
    pltpu.sync_copy(tp.at[wid], v_tp)
    pltpu.sync_copy(td.at[wid], v_td)

    def g0(c, buf, gsem):
        return pltpu.async_copy(tok.at[v_t0.at[c]], buf, gsem)

    def adds(c, buf, gsem):
        pltpu.async_copy(tok.at[v_t1.at[c]], buf, gsem, add=True)
        pltpu.async_copy(tok.at[v_t2.at[c]], buf, gsem, add=True)
        pltpu.async_copy(post.at[v_tp.at[c]], buf, gsem, add=True)

    def wait_adds(c, buf, gsem):
        pltpu.make_async_copy(tok.at[v_t1.at[c]], buf, gsem).wait()
        pltpu.make_async_copy(tok.at[v_t2.at[c]], buf, gsem).wait()
        pltpu.make_async_copy(post.at[v_tp.at[c]], buf, gsem).wait()

    def scat(c, buf, ssem):
        return pltpu.async_copy(buf, text_out.at[v_td.at[c]], ssem)

    def wait_scat(c, buf, ssem):
        pltpu.make_async_copy(buf, text_out.at[v_td.at[c]], ssem).wait()

    g0(0, bA, gsA).wait()
    adds(0, bA, gsA)
    g0(1, bB, gsB)

    def pair(h, _):
        a = 2 * h
        b = a + 1
        pltpu.make_async_copy(tok.at[v_t0.at[b]], bB, gsB).wait()
        adds(b, bB, gsB)
        wait_adds(a, bA, gsA)
        scat(a, bA, ssA)

        @pl.when(a + 2 < _TCHUNKS)
        def _():
            wait_scat(a, bA, ssA)
            g0(a + 2, bA, gsA).wait()
            adds(a + 2, bA, gsA)

        wait_adds(b, bB, gsB)
        scat(b, bB, ssB)

        @pl.when(b + 2 < _TCHUNKS)
        def _():
            wait_scat(b, bB, ssB)
            g0(b + 2, bB, gsB)

        return 0

    lax.fori_loop(0, _TCHUNKS // 2, pair, 0)
    wait_scat(_TCHUNKS - 2, bA, ssA)
    wait_scat(_TCHUNKS - 1, bB, ssB)


_sc_text = functools.partial(
    pl.kernel,
    out_type=jax.ShapeDtypeStruct((_NT, _EMB), jnp.float32),
    mesh=_MESH,
    compiler_params=_PARAMS,
    scratch_types=[
        pltpu.VMEM((_TCHUNKS, _C), jnp.int32),
        pltpu.VMEM((_TCHUNKS, _C), jnp.int32),
        pltpu.VMEM((_TCHUNKS, _C), jnp.int32),
        pltpu.VMEM((_TCHUNKS, _C), jnp.int32),
        pltpu.VMEM((_TCHUNKS, _C), jnp.int32),
        pltpu.VMEM((_C, _EMB), jnp.float32),
        pltpu.VMEM((_C, _EMB), jnp.float32),
        pltpu.SemaphoreType.DMA,
        pltpu.SemaphoreType.DMA,
        pltpu.SemaphoreType.DMA,
        pltpu.SemaphoreType.DMA,
    ],
)(_sc_text_body)


def _sc_img_body(tok, i0, i1, i2, img_out,
                 v_i0, v_i1, v_i2, bA, bB, gsA, gsB):
    wid = lax.axis_index("s") * _NC + lax.axis_index("c")

    pltpu.sync_copy(i0.at[wid], v_i0)
    pltpu.sync_copy(i1.at[wid], v_i1)
    pltpu.sync_copy(i2.at[wid], v_i2)

    def g0(c, buf, gsem):
        return pltpu.async_copy(tok.at[v_i0.at[c]], buf, gsem)

    def adds(c, buf, gsem):
        pltpu.async_copy(tok.at[v_i1.at[c]], buf, gsem, add=True)
        pltpu.async_copy(tok.at[v_i2.at[c]], buf, gsem, add=True)

    def wait_adds(c, buf, gsem):
        pltpu.make_async_copy(tok.at[v_i1.at[c]], buf, gsem).wait()
        pltpu.make_async_copy(tok.at[v_i2.at[c]], buf, gsem).wait()

    def out_at(c):
        return img_out.at[pl.ds(wid * (_ICHUNKS * _C) + c * _C, _C)]

    bufs = (bA, bB)
    sems = (gsA, gsB)
    g0(0, bA, gsA).wait()
    adds(0, bA, gsA)
    for c in range(1, _ICHUNKS + 1):
        if c < _ICHUNKS:
            g0(c, bufs[c % 2], sems[c % 2]).wait()
            adds(c, bufs[c % 2], sems[c % 2])
        p = c - 1
        wait_adds(p, bufs[p % 2], sems[p % 2])
        pltpu.sync_copy(bufs[p % 2], out_at(p))


_sc_img = functools.partial(
    pl.kernel,
    out_type=jax.ShapeDtypeStruct((_NI, _EMB), jnp.float32),
    mesh=_MESH,
    compiler_params=_PARAMS,
    scratch_types=[
        pltpu.VMEM((_ICHUNKS, _C), jnp.int32),
        pltpu.VMEM((_ICHUNKS, _C), jnp.int32),
        pltpu.VMEM((_ICHUNKS, _C), jnp.int32),
        pltpu.VMEM((_C, _EMB), jnp.float32),
        pltpu.VMEM((_C, _EMB), jnp.float32),
        pltpu.SemaphoreType.DMA,
        pltpu.SemaphoreType.DMA,
    ],
)(_sc_img_body)


def _mm_body(x_ref, w_ref, g_ref, o_ref):
    o_ref[...] = (
        jnp.dot(x_ref[...], w_ref[...], preferred_element_type=jnp.float32)
        + g_ref[...]
    )


def _img_project(x, w_t, gsum):
    m_blk = 1024
    return pl.pallas_call(
        _mm_body,
        grid=(_NI // m_blk,),
        in_specs=[
            pl.BlockSpec((m_blk, _IMG_IN), lambda i: (i, 0)),
            pl.BlockSpec((_IMG_IN, _EMB), lambda i: (0, 0)),
            pl.BlockSpec((m_blk, _EMB), lambda i: (i, 0)),
        ],
        out_specs=pl.BlockSpec((m_blk, _EMB), lambda i: (i, 0)),
        out_shape=jax.ShapeDtypeStruct((_NI, _EMB), jnp.float32),
    )(x, w_t, gsum)


def kernel(src_input, src_pos, src_turn, src_speaker, image_input,
           image_pos, image_turn, image_speaker, tok_table, pos_table, W_img):
    i32 = jnp.int32

    def lmajor(a, chunks):
        return a.T.astype(i32).reshape(_NW, chunks, _C)

    t0 = lmajor(src_input, _TCHUNKS)
    t1 = lmajor(src_turn, _TCHUNKS)
    t2 = lmajor(src_speaker, _TCHUNKS)
    tp = lmajor(src_pos, _TCHUNKS)
    i0 = lmajor(image_turn, _ICHUNKS)
    i1 = lmajor(image_speaker, _ICHUNKS)
    i2 = lmajor(image_pos, _ICHUNKS)

    p = jnp.arange(_NT, dtype=i32)
    td = ((p % _BT) * _LT + p // _BT).reshape(_NW, _TCHUNKS, _C)

    img_gather = _sc_img(tok_table, i0, i1, i2)
    text_flat = _sc_text(tok_table, pos_table, t0, t1, t2, tp, td)

    x = image_input.transpose(1, 0, 2).reshape(_NI, _IMG_IN)
    img_flat = _img_project(x, W_img.T, img_gather)

    return (text_flat.reshape(_BT, _LT, _EMB),
            img_flat.reshape(_LI, _BT, _EMB).transpose(1, 0, 2))


# speedup vs baseline: 8.0257x; 1.0225x over previous
import functools

import jax
import jax.numpy as jnp
from jax import lax
from jax.experimental import pallas as pl
from jax.experimental.pallas import tpu as pltpu
from jax.experimental.pallas import tpu_sc as plsc

_EMB = 128
_NC, _NS = 2, 16
_NW = _NC * _NS
_C = 128

_BT, _LT = 1024, 200
_LI = 20
_NT = _BT * _LT
_NI = _BT * _LI
_TCHUNKS = _NT // (_NW * _C)
_ICHUNKS = _NI // (_NW * _C)
_IMG_IN = 2048

_MESH = plsc.VectorSubcoreMesh(core_axis_name="c", subcore_axis_name="s")
_PARAMS = pltpu.CompilerParams(use_tc_tiling_on_sc=True)


def _sc_text_body(tok, post, t0, t1, t2, tp, td, text_out,
                  v_t0, v_t1, v_t2, v_tp, v_td,
                  bA, bB, bC, gsA, gsB, gsC, ssA, ssB, ssC):
    wid = lax.axis_index("s") * _NC + lax.axis_index("c")

    pltpu.sync_copy(t0.at[wid], v_t0)
    pltpu.sync_copy(t1.at[wid], v_t1)
    pltpu.sync_copy(t2.at[wid], v_t2)
    pltpu.sync_copy(tp.at[wid], v_tp)
    pltpu.sync_copy(td.at[wid], v_td)

    def g0(c, buf, gsem):
        return pltpu.async_copy(tok.at[v_t0.at[c]], buf, gsem)

    def adds(c, buf, gsem):
        pltpu.async_copy(tok.at[v_t1.at[c]], buf, gsem, add=True)
        pltpu.async_copy(tok.at[v_t2.at[c]], buf, gsem, add=True)
        pltpu.async_copy(post.at[v_tp.at[c]], buf, gsem, add=True)

    def wait_adds(c, buf, gsem):
        pltpu.make_async_copy(tok.at[v_t1.at[c]], buf, gsem).wait()
        pltpu.make_async_copy(tok.at[v_t2.at[c]], buf, gsem).wait()
        pltpu.make_async_copy(post.at[v_tp.at[c]], buf, gsem).wait()

    def scat(c, buf, ssem):
        return pltpu.async_copy(buf, text_out.at[v_td.at[c]], ssem)

    def wait_scat(c, buf, ssem):
        pltpu.make_async_copy(buf, text_out.at[v_td.at[c]], ssem).wait()

    bufs = (bA, bB, bC)
    gsems = (gsA, gsB, gsC)
    ssems = (ssA, ssB, ssC)
    n_slots = _TCHUNKS + 2
    n_iters = (n_slots + 2) // 3

    def ring(i, _):
        for k in range(3):
            c = 3 * i + k
            buf, gsem, ssem = bufs[k], gsems[k], ssems[k]

            @pl.when(jnp.logical_and(c >= 3, c < _TCHUNKS))
            def _():
                wait_scat(c - 3, buf, ssem)

            @pl.when(c < _TCHUNKS)
            def _():
                g0(c, buf, gsem)

            p = c - 1
            pbuf, pgsem = bufs[(k + 2) % 3], gsems[(k + 2) % 3]

            @pl.when(jnp.logical_and(p >= 0, p < _TCHUNKS))
            def _():
                pltpu.make_async_copy(tok.at[v_t0.at[p]], pbuf, pgsem).wait()
                adds(p, pbuf, pgsem)

            q = c - 2
            qbuf, qgsem, qssem = (bufs[(k + 1) % 3], gsems[(k + 1) % 3],
                                  ssems[(k + 1) % 3])

            @pl.when(jnp.logical_and(q >= 0, q < _TCHUNKS))
            def _():
                wait_adds(q, qbuf, qgsem)
                scat(q, qbuf, qssem)

        return 0

    lax.fori_loop(0, n_iters, ring, 0)
    for c in (_TCHUNKS - 3, _TCHUNKS - 2, _TCHUNKS - 1):
        wait_scat(c, bufs[c % 3], ssems[c % 3])


_sc_text = functools.partial(
    pl.kernel,
    out_type=jax.ShapeDtypeStruct((_NT, _EMB), jnp.float32),
    mesh=_MESH,
    compiler_params=_PARAMS,
    scratch_types=[
        pltpu.VMEM((_TCHUNKS, _C), jnp.int32),
        pltpu.VMEM((_TCHUNKS, _C), jnp.int32),
        pltpu.VMEM((_TCHUNKS, _C), jnp.int32),
        pltpu.VMEM((_TCHUNKS, _C), jnp.int32),
        pltpu.VMEM((_TCHUNKS, _C), jnp.int32),
        pltpu.VMEM((_C, _EMB), jnp.float32),
        pltpu.VMEM((_C, _EMB), jnp.float32),
        pltpu.VMEM((_C, _EMB), jnp.float32),
        pltpu.SemaphoreType.DMA,
        pltpu.SemaphoreType.DMA,
        pltpu.SemaphoreType.DMA,
        pltpu.SemaphoreType.DMA,
        pltpu.SemaphoreType.DMA,
        pltpu.SemaphoreType.DMA,
    ],
)(_sc_text_body)


def _sc_img_body(tok, i0, i1, i2, img_out,
                 v_i0, v_i1, v_i2, bA, bB, gsA, gsB):
    wid = lax.axis_index("s") * _NC + lax.axis_index("c")

    pltpu.sync_copy(i0.at[wid], v_i0)
    pltpu.sync_copy(i1.at[wid], v_i1)
    pltpu.sync_copy(i2.at[wid], v_i2)

    def g0(c, buf, gsem):
        return pltpu.async_copy(tok.at[v_i0.at[c]], buf, gsem)

    def adds(c, buf, gsem):
        pltpu.async_copy(tok.at[v_i1.at[c]], buf, gsem, add=True)
        pltpu.async_copy(tok.at[v_i2.at[c]], buf, gsem, add=True)

    def wait_adds(c, buf, gsem):
        pltpu.make_async_copy(tok.at[v_i1.at[c]], buf, gsem).wait()
        pltpu.make_async_copy(tok.at[v_i2.at[c]], buf, gsem).wait()

    def out_at(c):
        return img_out.at[pl.ds(wid * (_ICHUNKS * _C) + c * _C, _C)]

    bufs = (bA, bB)
    sems = (gsA, gsB)
    g0(0, bA, gsA).wait()
    adds(0, bA, gsA)
    for c in range(1, _ICHUNKS + 1):
        if c < _ICHUNKS:
            g0(c, bufs[c % 2], sems[c % 2]).wait()
            adds(c, bufs[c % 2], sems[c % 2])
        p = c - 1
        wait_adds(p, bufs[p % 2], sems[p % 2])
        pltpu.sync_copy(bufs[p % 2], out_at(p))


_sc_img = functools.partial(
    pl.kernel,
    out_type=jax.ShapeDtypeStruct((_NI, _EMB), jnp.float32),
    mesh=_MESH,
    compiler_params=_PARAMS,
    scratch_types=[
        pltpu.VMEM((_ICHUNKS, _C), jnp.int32),
        pltpu.VMEM((_ICHUNKS, _C), jnp.int32),
        pltpu.VMEM((_ICHUNKS, _C), jnp.int32),
        pltpu.VMEM((_C, _EMB), jnp.float32),
        pltpu.VMEM((_C, _EMB), jnp.float32),
        pltpu.SemaphoreType.DMA,
        pltpu.SemaphoreType.DMA,
    ],
)(_sc_img_body)


def _mm_body(x_ref, w_ref, g_ref, o_ref):
    o_ref[...] = (
        jnp.dot(x_ref[...], w_ref[...], preferred_element_type=jnp.float32)
        + g_ref[...]
    )


def _img_project(x, w_t, gsum):
    m_blk = 1024
    return pl.pallas_call(
        _mm_body,
        grid=(_NI // m_blk,),
        in_specs=[
            pl.BlockSpec((m_blk, _IMG_IN), lambda i: (i, 0)),
            pl.BlockSpec((_IMG_IN, _EMB), lambda i: (0, 0)),
            pl.BlockSpec((m_blk, _EMB), lambda i: (i, 0)),
        ],
        out_specs=pl.BlockSpec((m_blk, _EMB), lambda i: (i, 0)),
        out_shape=jax.ShapeDtypeStruct((_NI, _EMB), jnp.float32),
    )(x, w_t, gsum)


def kernel(src_input, src_pos, src_turn, src_speaker, image_input,
           image_pos, image_turn, image_speaker, tok_table, pos_table, W_img):
    i32 = jnp.int32

    def lmajor(a, chunks):
        return a.T.astype(i32).reshape(_NW, chunks, _C)

    t0 = lmajor(src_input, _TCHUNKS)
    t1 = lmajor(src_turn, _TCHUNKS)
    t2 = lmajor(src_speaker, _TCHUNKS)
    tp = lmajor(src_pos, _TCHUNKS)
    i0 = lmajor(image_turn, _ICHUNKS)
    i1 = lmajor(image_speaker, _ICHUNKS)
    i2 = lmajor(image_pos, _ICHUNKS)

    p = jnp.arange(_NT, dtype=i32)
    td = ((p % _BT) * _LT + p // _BT).reshape(_NW, _TCHUNKS, _C)

    img_gather = _sc_img(tok_table, i0, i1, i2)
    text_flat = _sc_text(tok_table, pos_table, t0, t1, t2, tp, td)

    x = image_input.transpose(1, 0, 2).reshape(_NI, _IMG_IN)
    img_flat = _img_project(x, W_img.T, img_gather)

    return (text_flat.reshape(_BT, _LT, _EMB),
            img_flat.reshape(_LI, _BT, _EMB).transpose(1, 0, 2))


# speedup vs baseline: 8.4562x; 1.0536x over previous
import functools

import jax
import jax.numpy as jnp
from jax import lax
from jax.experimental import pallas as pl
from jax.experimental.pallas import tpu as pltpu
from jax.experimental.pallas import tpu_sc as plsc

_EMB = 128
_NC, _NS = 2, 16
_NW = _NC * _NS
_C = 128

_BT, _LT = 1024, 200
_LI = 20
_NT = _BT * _LT
_NI = _BT * _LI
_TCHUNKS = _NT // (_NW * _C)
_ICHUNKS = _NI // (_NW * _C)
_IMG_IN = 2048

_MESH = plsc.VectorSubcoreMesh(core_axis_name="c", subcore_axis_name="s")
_PARAMS = pltpu.CompilerParams(use_tc_tiling_on_sc=True)


def _sc_text_body(tok, post, t0, t1, t2, tp, td, text_out,
                  v_t0, v_t1, v_t2, v_tp, v_td, pv,
                  bA, bB, bC, gsA, gsB, gsC, ssA, ssB, ssC):
    wid = lax.axis_index("s") * _NC + lax.axis_index("c")

    pltpu.sync_copy(t0.at[wid], v_t0)
    pltpu.sync_copy(t1.at[wid], v_t1)
    pltpu.sync_copy(t2.at[wid], v_t2)
    pltpu.sync_copy(tp.at[wid], v_tp)
    pltpu.sync_copy(td.at[wid], v_td)
    pltpu.sync_copy(post, pv)

    def g0(c, buf, gsem):
        return pltpu.async_copy(tok.at[v_t0.at[c]], buf, gsem)

    def adds(c, buf, gsem):
        pltpu.async_copy(tok.at[v_t1.at[c]], buf, gsem, add=True)
        pltpu.async_copy(tok.at[v_t2.at[c]], buf, gsem, add=True)

    def wait_adds(c, buf, gsem):
        pltpu.make_async_copy(tok.at[v_t1.at[c]], buf, gsem).wait()
        pltpu.make_async_copy(tok.at[v_t2.at[c]], buf, gsem).wait()

    def pos_add(c, buf):
        def group(g, _):
            base = g * 16
            idx16 = v_tp[c, pl.ds(base, 16)]
            for r in range(16):
                j = idx16[r]
                for m in range(_EMB // 16):
                    s = pl.ds(m * 16, 16)
                    buf[base + r, s] = buf[base + r, s] + pv[j, s]
            return 0

        lax.fori_loop(0, _C // 16, group, 0)

    def scat(c, buf, ssem):
        return pltpu.async_copy(buf, text_out.at[v_td.at[c]], ssem)

    def wait_scat(c, buf, ssem):
        pltpu.make_async_copy(buf, text_out.at[v_td.at[c]], ssem).wait()

    bufs = (bA, bB, bC)
    gsems = (gsA, gsB, gsC)
    ssems = (ssA, ssB, ssC)
    n_slots = _TCHUNKS + 2
    n_iters = (n_slots + 2) // 3

    def ring(i, _):
        for k in range(3):
            c = 3 * i + k
            buf, gsem, ssem = bufs[k], gsems[k], ssems[k]

            @pl.when(jnp.logical_and(c >= 3, c < _TCHUNKS))
            def _():
                wait_scat(c - 3, buf, ssem)

            @pl.when(c < _TCHUNKS)
            def _():
                g0(c, buf, gsem)

            p = c - 1
            pbuf, pgsem = bufs[(k + 2) % 3], gsems[(k + 2) % 3]

            @pl.when(jnp.logical_and(p >= 0, p < _TCHUNKS))
            def _():
                pltpu.make_async_copy(tok.at[v_t0.at[p]], pbuf, pgsem).wait()
                adds(p, pbuf, pgsem)

            q = c - 2
            qbuf, qgsem, qssem = (bufs[(k + 1) % 3], gsems[(k + 1) % 3],
                                  ssems[(k + 1) % 3])

            @pl.when(jnp.logical_and(q >= 0, q < _TCHUNKS))
            def _():
                wait_adds(q, qbuf, qgsem)
                pos_add(q, qbuf)
                scat(q, qbuf, qssem)

        return 0

    lax.fori_loop(0, n_iters, ring, 0)
    for c in (_TCHUNKS - 3, _TCHUNKS - 2, _TCHUNKS - 1):
        wait_scat(c, bufs[c % 3], ssems[c % 3])


_sc_text = functools.partial(
    pl.kernel,
    out_type=jax.ShapeDtypeStruct((_NT, _EMB), jnp.float32),
    mesh=_MESH,
    compiler_params=_PARAMS,
    scratch_types=[
        pltpu.VMEM((_TCHUNKS, _C), jnp.int32),
        pltpu.VMEM((_TCHUNKS, _C), jnp.int32),
        pltpu.VMEM((_TCHUNKS, _C), jnp.int32),
        pltpu.VMEM((_TCHUNKS, _C), jnp.int32),
        pltpu.VMEM((_TCHUNKS, _C), jnp.int32),
        pltpu.VMEM((_LT + 1, _EMB), jnp.float32),
        pltpu.VMEM((_C, _EMB), jnp.float32),
        pltpu.VMEM((_C, _EMB), jnp.float32),
        pltpu.VMEM((_C, _EMB), jnp.float32),
        pltpu.SemaphoreType.DMA,
        pltpu.SemaphoreType.DMA,
        pltpu.SemaphoreType.DMA,
        pltpu.SemaphoreType.DMA,
        pltpu.SemaphoreType.DMA,
        pltpu.SemaphoreType.DMA,
    ],
)(_sc_text_body)


def _sc_img_body(tok, i0, i1, i2, img_out,
                 v_i0, v_i1, v_i2, bA, bB, gsA, gsB):
    wid = lax.axis_index("s") * _NC + lax.axis_index("c")

    pltpu.sync_copy(i0.at[wid], v_i0)
    pltpu.sync_copy(i1.at[wid], v_i1)
    pltpu.sync_copy(i2.at[wid], v_i2)

    def g0(c, buf, gsem):
        return pltpu.async_copy(tok.at[v_i0.at[c]], buf, gsem)

    def adds(c, buf, gsem):
        pltpu.async_copy(tok.at[v_i1.at[c]], buf, gsem, add=True)
        pltpu.async_copy(tok.at[v_i2.at[c]], buf, gsem, add=True)

    def wait_adds(c, buf, gsem):
        pltpu.make_async_copy(tok.at[v_i1.at[c]], buf, gsem).wait()
        pltpu.make_async_copy(tok.at[v_i2.at[c]], buf, gsem).wait()

    def out_at(c):
        return img_out.at[pl.ds(wid * (_ICHUNKS * _C) + c * _C, _C)]

    bufs = (bA, bB)
    sems = (gsA, gsB)
    g0(0, bA, gsA).wait()
    adds(0, bA, gsA)
    for c in range(1, _ICHUNKS + 1):
        if c < _ICHUNKS:
            g0(c, bufs[c % 2], sems[c % 2]).wait()
            adds(c, bufs[c % 2], sems[c % 2])
        p = c - 1
        wait_adds(p, bufs[p % 2], sems[p % 2])
        pltpu.sync_copy(bufs[p % 2], out_at(p))


_sc_img = functools.partial(
    pl.kernel,
    out_type=jax.ShapeDtypeStruct((_NI, _EMB), jnp.float32),
    mesh=_MESH,
    compiler_params=_PARAMS,
    scratch_types=[
        pltpu.VMEM((_ICHUNKS, _C), jnp.int32),
        pltpu.VMEM((_ICHUNKS, _C), jnp.int32),
        pltpu.VMEM((_ICHUNKS, _C), jnp.int32),
        pltpu.VMEM((_C, _EMB), jnp.float32),
        pltpu.VMEM((_C, _EMB), jnp.float32),
        pltpu.SemaphoreType.DMA,
        pltpu.SemaphoreType.DMA,
    ],
)(_sc_img_body)


def _mm_body(x_ref, w_ref, g_ref, o_ref):
    o_ref[...] = (
        jnp.dot(x_ref[...], w_ref[...], preferred_element_type=jnp.float32)
        + g_ref[...]
    )


def _img_project(x, w_t, gsum):
    m_blk = 1024
    return pl.pallas_call(
        _mm_body,
        grid=(_NI // m_blk,),
        in_specs=[
            pl.BlockSpec((m_blk, _IMG_IN), lambda i: (i, 0)),
            pl.BlockSpec((_IMG_IN, _EMB), lambda i: (0, 0)),
            pl.BlockSpec((m_blk, _EMB), lambda i: (i, 0)),
        ],
        out_specs=pl.BlockSpec((m_blk, _EMB), lambda i: (i, 0)),
        out_shape=jax.ShapeDtypeStruct((_NI, _EMB), jnp.float32),
    )(x, w_t, gsum)


def kernel(src_input, src_pos, src_turn, src_speaker, image_input,
           image_pos, image_turn, image_speaker, tok_table, pos_table, W_img):
    i32 = jnp.int32

    def lmajor(a, chunks):
        return a.T.astype(i32).reshape(_NW, chunks, _C)

    t0 = lmajor(src_input, _TCHUNKS)
    t1 = lmajor(src_turn, _TCHUNKS)
    t2 = lmajor(src_speaker, _TCHUNKS)
    tp = lmajor(src_pos, _TCHUNKS)
    i0 = lmajor(image_turn, _ICHUNKS)
    i1 = lmajor(image_speaker, _ICHUNKS)
    i2 = lmajor(image_pos, _ICHUNKS)

    p = jnp.arange(_NT, dtype=i32)
    td = ((p % _BT) * _LT + p // _BT).reshape(_NW, _TCHUNKS, _C)

    img_gather = _sc_img(tok_table, i0, i1, i2)
    text_flat = _sc_text(tok_table, pos_table, t0, t1, t2, tp, td)

    x = image_input.transpose(1, 0, 2).reshape(_NI, _IMG_IN)
    img_flat = _img_project(x, W_img.T, img_gather)

    return (text_flat.reshape(_BT, _LT, _EMB),
            img_flat.reshape(_LI, _BT, _EMB).transpose(1, 0, 2))


# speedup vs baseline: 10.8398x; 1.2819x over previous
import functools

import jax
import jax.numpy as jnp
from jax import lax
from jax.experimental import pallas as pl
from jax.experimental.pallas import tpu as pltpu
from jax.experimental.pallas import tpu_sc as plsc

_EMB = 128
_NC, _NS = 2, 16
_NW = _NC * _NS
_C = 128

_BT, _LT = 1024, 200
_LI = 20
_NT = _BT * _LT
_NI = _BT * _LI
_TCHUNKS = _NT // (_NW * _C)
_ICHUNKS = _NI // (_NW * _C)
_IMG_IN = 2048

_MESH = plsc.VectorSubcoreMesh(core_axis_name="c", subcore_axis_name="s")
_PARAMS = pltpu.CompilerParams(use_tc_tiling_on_sc=True)


def _sc_text_body(tok, post, t0, t1, t2, tp, td, text_out,
                  v_t0, v_t1, v_t2, v_tp, v_td, pv,
                  bA, bB, bC, gsA, gsB, gsC, ssA, ssB, ssC):
    wid = lax.axis_index("s") * _NC + lax.axis_index("c")

    pltpu.sync_copy(t0.at[wid], v_t0)
    pltpu.sync_copy(t1.at[wid], v_t1)
    pltpu.sync_copy(t2.at[wid], v_t2)
    pltpu.sync_copy(tp.at[wid], v_tp)
    pltpu.sync_copy(td.at[wid], v_td)
    pltpu.sync_copy(post, pv)

    def adds3(c, buf, gsem):
        pltpu.async_copy(tok.at[v_t0.at[c]], buf, gsem, add=True)
        pltpu.async_copy(tok.at[v_t1.at[c]], buf, gsem, add=True)
        pltpu.async_copy(tok.at[v_t2.at[c]], buf, gsem, add=True)

    def wait_adds3(c, buf, gsem):
        pltpu.make_async_copy(tok.at[v_t0.at[c]], buf, gsem).wait()
        pltpu.make_async_copy(tok.at[v_t1.at[c]], buf, gsem).wait()
        pltpu.make_async_copy(tok.at[v_t2.at[c]], buf, gsem).wait()

    def pos_fill(c, buf):
        def group(g, _):
            base = g * 16
            idx16 = v_tp[c, pl.ds(base, 16)]
            for r in range(16):
                j = idx16[r]
                for m in range(_EMB // 16):
                    s = pl.ds(m * 16, 16)
                    buf[base + r, s] = pv[j, s]
            return 0

        lax.fori_loop(0, _C // 16, group, 0)

    def scat(c, buf, ssem):
        return pltpu.async_copy(buf, text_out.at[v_td.at[c]], ssem)

    def wait_scat(c, buf, ssem):
        pltpu.make_async_copy(buf, text_out.at[v_td.at[c]], ssem).wait()

    bufs = (bA, bB, bC)
    gsems = (gsA, gsB, gsC)
    ssems = (ssA, ssB, ssC)
    n_slots = _TCHUNKS + 1
    n_iters = (n_slots + 2) // 3

    def ring(i, _):
        for k in range(3):
            c = 3 * i + k
            buf, gsem, ssem = bufs[k], gsems[k], ssems[k]

            @pl.when(jnp.logical_and(c >= 3, c < _TCHUNKS))
            def _():
                wait_scat(c - 3, buf, ssem)

            @pl.when(c < _TCHUNKS)
            def _():
                pos_fill(c, buf)
                adds3(c, buf, gsem)

            p = c - 1
            pbuf, pgsem, pssem = (bufs[(k + 2) % 3], gsems[(k + 2) % 3],
                                  ssems[(k + 2) % 3])

            @pl.when(jnp.logical_and(p >= 0, p < _TCHUNKS))
            def _():
                wait_adds3(p, pbuf, pgsem)
                scat(p, pbuf, pssem)

        return 0

    lax.fori_loop(0, n_iters, ring, 0)
    for c in (_TCHUNKS - 3, _TCHUNKS - 2, _TCHUNKS - 1):
        wait_scat(c, bufs[c % 3], ssems[c % 3])


_sc_text = functools.partial(
    pl.kernel,
    out_type=jax.ShapeDtypeStruct((_NT, _EMB), jnp.float32),
    mesh=_MESH,
    compiler_params=_PARAMS,
    scratch_types=[
        pltpu.VMEM((_TCHUNKS, _C), jnp.int32),
        pltpu.VMEM((_TCHUNKS, _C), jnp.int32),
        pltpu.VMEM((_TCHUNKS, _C), jnp.int32),
        pltpu.VMEM((_TCHUNKS, _C), jnp.int32),
        pltpu.VMEM((_TCHUNKS, _C), jnp.int32),
        pltpu.VMEM((_LT + 1, _EMB), jnp.float32),
        pltpu.VMEM((_C, _EMB), jnp.float32),
        pltpu.VMEM((_C, _EMB), jnp.float32),
        pltpu.VMEM((_C, _EMB), jnp.float32),
        pltpu.SemaphoreType.DMA,
        pltpu.SemaphoreType.DMA,
        pltpu.SemaphoreType.DMA,
        pltpu.SemaphoreType.DMA,
        pltpu.SemaphoreType.DMA,
        pltpu.SemaphoreType.DMA,
    ],
)(_sc_text_body)


def _sc_img_body(tok, i0, i1, i2, img_out,
                 v_i0, v_i1, v_i2, bA, bB, gsA, gsB):
    wid = lax.axis_index("s") * _NC + lax.axis_index("c")

    pltpu.sync_copy(i0.at[wid], v_i0)
    pltpu.sync_copy(i1.at[wid], v_i1)
    pltpu.sync_copy(i2.at[wid], v_i2)

    def g0(c, buf, gsem):
        return pltpu.async_copy(tok.at[v_i0.at[c]], buf, gsem)

    def adds(c, buf, gsem):
        pltpu.async_copy(tok.at[v_i1.at[c]], buf, gsem, add=True)
        pltpu.async_copy(tok.at[v_i2.at[c]], buf, gsem, add=True)

    def wait_adds(c, buf, gsem):
        pltpu.make_async_copy(tok.at[v_i1.at[c]], buf, gsem).wait()
        pltpu.make_async_copy(tok.at[v_i2.at[c]], buf, gsem).wait()

    def out_at(c):
        return img_out.at[pl.ds(wid * (_ICHUNKS * _C) + c * _C, _C)]

    bufs = (bA, bB)
    sems = (gsA, gsB)
    g0(0, bA, gsA).wait()
    adds(0, bA, gsA)
    for c in range(1, _ICHUNKS + 1):
        if c < _ICHUNKS:
            g0(c, bufs[c % 2], sems[c % 2]).wait()
            adds(c, bufs[c % 2], sems[c % 2])
        p = c - 1
        wait_adds(p, bufs[p % 2], sems[p % 2])
        pltpu.sync_copy(bufs[p % 2], out_at(p))


_sc_img = functools.partial(
    pl.kernel,
    out_type=jax.ShapeDtypeStruct((_NI, _EMB), jnp.float32),
    mesh=_MESH,
    compiler_params=_PARAMS,
    scratch_types=[
        pltpu.VMEM((_ICHUNKS, _C), jnp.int32),
        pltpu.VMEM((_ICHUNKS, _C), jnp.int32),
        pltpu.VMEM((_ICHUNKS, _C), jnp.int32),
        pltpu.VMEM((_C, _EMB), jnp.float32),
        pltpu.VMEM((_C, _EMB), jnp.float32),
        pltpu.SemaphoreType.DMA,
        pltpu.SemaphoreType.DMA,
    ],
)(_sc_img_body)


def _mm_body(x_ref, w_ref, g_ref, o_ref):
    o_ref[...] = (
        jnp.dot(x_ref[...], w_ref[...], preferred_element_type=jnp.float32)
        + g_ref[...]
    )


def _img_project(x, w_t, gsum):
    m_blk = 1024
    return pl.pallas_call(
        _mm_body,
        grid=(_NI // m_blk,),
        in_specs=[
            pl.BlockSpec((m_blk, _IMG_IN), lambda i: (i, 0)),
            pl.BlockSpec((_IMG_IN, _EMB), lambda i: (0, 0)),
            pl.BlockSpec((m_blk, _EMB), lambda i: (i, 0)),
        ],
        out_specs=pl.BlockSpec((m_blk, _EMB), lambda i: (i, 0)),
        out_shape=jax.ShapeDtypeStruct((_NI, _EMB), jnp.float32),
    )(x, w_t, gsum)


def kernel(src_input, src_pos, src_turn, src_speaker, image_input,
           image_pos, image_turn, image_speaker, tok_table, pos_table, W_img):
    i32 = jnp.int32

    def lmajor(a, chunks):
        return a.T.astype(i32).reshape(_NW, chunks, _C)

    t0 = lmajor(src_input, _TCHUNKS)
    t1 = lmajor(src_turn, _TCHUNKS)
    t2 = lmajor(src_speaker, _TCHUNKS)
    tp = lmajor(src_pos, _TCHUNKS)
    i0 = lmajor(image_turn, _ICHUNKS)
    i1 = lmajor(image_speaker, _ICHUNKS)
    i2 = lmajor(image_pos, _ICHUNKS)

    p = jnp.arange(_NT, dtype=i32)
    td = ((p % _BT) * _LT + p // _BT).reshape(_NW, _TCHUNKS, _C)

    img_gather = _sc_img(tok_table, i0, i1, i2)
    text_flat = _sc_text(tok_table, pos_table, t0, t1, t2, tp, td)

    x = image_input.transpose(1, 0, 2).reshape(_NI, _IMG_IN)
    img_flat = _img_project(x, W_img.T, img_gather)

    return (text_flat.reshape(_BT, _LT, _EMB),
            img_flat.reshape(_LI, _BT, _EMB).transpose(1, 0, 2))


# speedup vs baseline: 10.8784x; 1.0036x over previous
import functools

import jax
import jax.numpy as jnp
from jax import lax
from jax.experimental import pallas as pl
from jax.experimental.pallas import tpu as pltpu
from jax.experimental.pallas import tpu_sc as plsc

_EMB = 128
_NC, _NS = 2, 16
_NW = _NC * _NS
_C = 128

_BT, _LT = 1024, 200
_LI = 20
_NT = _BT * _LT
_NI = _BT * _LI
_TCHUNKS = _NT // (_NW * _C)
_ICHUNKS = _NI // (_NW * _C)
_IMG_IN = 2048

_MESH = plsc.VectorSubcoreMesh(core_axis_name="c", subcore_axis_name="s")
_PARAMS = pltpu.CompilerParams(use_tc_tiling_on_sc=True)


def _sc_text_body(tok, post, t0, t1, t2, tp, text_out,
                  v_t0, v_t1, v_t2, v_tp, didx, pv,
                  bA, bB, bC, gsA, gsB, gsC, ssA, ssB, ssC):
    wid = lax.axis_index("s") * _NC + lax.axis_index("c")

    pltpu.sync_copy(t0.at[wid], v_t0)
    pltpu.sync_copy(t1.at[wid], v_t1)
    pltpu.sync_copy(t2.at[wid], v_t2)
    pltpu.sync_copy(tp.at[wid], v_tp)
    pltpu.sync_copy(post, pv)

    lane = jnp.arange(16, dtype=jnp.int32) * _LT

    def dst_fill(c, k):
        gg = wid * _TCHUNKS + c
        base = (gg % (_BT // _C)) * _C * _LT + gg // (_BT // _C)
        for g in range(_C // 16):
            didx[k, pl.ds(g * 16, 16)] = lane + (base + g * 16 * _LT)

    def adds3(c, buf, gsem):
        pltpu.async_copy(tok.at[v_t0.at[c]], buf, gsem, add=True)
        pltpu.async_copy(tok.at[v_t1.at[c]], buf, gsem, add=True)
        pltpu.async_copy(tok.at[v_t2.at[c]], buf, gsem, add=True)

    def wait_adds3(c, buf, gsem):
        pltpu.make_async_copy(tok.at[v_t0.at[c]], buf, gsem).wait()
        pltpu.make_async_copy(tok.at[v_t1.at[c]], buf, gsem).wait()
        pltpu.make_async_copy(tok.at[v_t2.at[c]], buf, gsem).wait()

    def pos_fill(c, buf):
        def group(g, _):
            base = g * 16
            idx16 = v_tp[c, pl.ds(base, 16)]
            for r in range(16):
                j = idx16[r]
                for m in range(_EMB // 16):
                    s = pl.ds(m * 16, 16)
                    buf[base + r, s] = pv[j, s]
            return 0

        lax.fori_loop(0, _C // 16, group, 0)

    def scat(k, buf, ssem):
        return pltpu.async_copy(buf, text_out.at[didx.at[k]], ssem)

    def wait_scat(k, buf, ssem):
        pltpu.make_async_copy(buf, text_out.at[didx.at[k]], ssem).wait()

    bufs = (bA, bB, bC)
    gsems = (gsA, gsB, gsC)
    ssems = (ssA, ssB, ssC)
    n_slots = _TCHUNKS + 1
    n_iters = (n_slots + 2) // 3

    def ring(i, _):
        for k in range(3):
            c = 3 * i + k
            buf, gsem, ssem = bufs[k], gsems[k], ssems[k]

            @pl.when(jnp.logical_and(c >= 3, c < _TCHUNKS))
            def _():
                wait_scat(k, buf, ssem)

            @pl.when(c < _TCHUNKS)
            def _():
                dst_fill(c, k)
                pos_fill(c, buf)
                adds3(c, buf, gsem)

            p = c - 1
            kp = (k + 2) % 3
            pbuf, pgsem, pssem = bufs[kp], gsems[kp], ssems[kp]

            @pl.when(jnp.logical_and(p >= 0, p < _TCHUNKS))
            def _():
                wait_adds3(p, pbuf, pgsem)
                scat(kp, pbuf, pssem)

        return 0

    lax.fori_loop(0, n_iters, ring, 0)
    for c in (_TCHUNKS - 3, _TCHUNKS - 2, _TCHUNKS - 1):
        wait_scat(c % 3, bufs[c % 3], ssems[c % 3])


_sc_text = functools.partial(
    pl.kernel,
    out_type=jax.ShapeDtypeStruct((_NT, _EMB), jnp.float32),
    mesh=_MESH,
    compiler_params=_PARAMS,
    scratch_types=[
        pltpu.VMEM((_TCHUNKS, _C), jnp.int32),
        pltpu.VMEM((_TCHUNKS, _C), jnp.int32),
        pltpu.VMEM((_TCHUNKS, _C), jnp.int32),
        pltpu.VMEM((_TCHUNKS, _C), jnp.int32),
        pltpu.VMEM((3, _C), jnp.int32),
        pltpu.VMEM((_LT + 1, _EMB), jnp.float32),
        pltpu.VMEM((_C, _EMB), jnp.float32),
        pltpu.VMEM((_C, _EMB), jnp.float32),
        pltpu.VMEM((_C, _EMB), jnp.float32),
        pltpu.SemaphoreType.DMA,
        pltpu.SemaphoreType.DMA,
        pltpu.SemaphoreType.DMA,
        pltpu.SemaphoreType.DMA,
        pltpu.SemaphoreType.DMA,
        pltpu.SemaphoreType.DMA,
    ],
)(_sc_text_body)


def _sc_img_body(tok, i0, i1, i2, img_out,
                 v_i0, v_i1, v_i2, bA, bB, gsA, gsB):
    wid = lax.axis_index("s") * _NC + lax.axis_index("c")

    pltpu.sync_copy(i0.at[wid], v_i0)
    pltpu.sync_copy(i1.at[wid], v_i1)
    pltpu.sync_copy(i2.at[wid], v_i2)

    def g0(c, buf, gsem):
        return pltpu.async_copy(tok.at[v_i0.at[c]], buf, gsem)

    def adds(c, buf, gsem):
        pltpu.async_copy(tok.at[v_i1.at[c]], buf, gsem, add=True)
        pltpu.async_copy(tok.at[v_i2.at[c]], buf, gsem, add=True)

    def wait_adds(c, buf, gsem):
        pltpu.make_async_copy(tok.at[v_i1.at[c]], buf, gsem).wait()
        pltpu.make_async_copy(tok.at[v_i2.at[c]], buf, gsem).wait()

    def out_at(c):
        return img_out.at[pl.ds(wid * (_ICHUNKS * _C) + c * _C, _C)]

    bufs = (bA, bB)
    sems = (gsA, gsB)
    g0(0, bA, gsA).wait()
    adds(0, bA, gsA)
    for c in range(1, _ICHUNKS + 1):
        if c < _ICHUNKS:
            g0(c, bufs[c % 2], sems[c % 2]).wait()
            adds(c, bufs[c % 2], sems[c % 2])
        p = c - 1
        wait_adds(p, bufs[p % 2], sems[p % 2])
        pltpu.sync_copy(bufs[p % 2], out_at(p))


_sc_img = functools.partial(
    pl.kernel,
    out_type=jax.ShapeDtypeStruct((_NI, _EMB), jnp.float32),
    mesh=_MESH,
    compiler_params=_PARAMS,
    scratch_types=[
        pltpu.VMEM((_ICHUNKS, _C), jnp.int32),
        pltpu.VMEM((_ICHUNKS, _C), jnp.int32),
        pltpu.VMEM((_ICHUNKS, _C), jnp.int32),
        pltpu.VMEM((_C, _EMB), jnp.float32),
        pltpu.VMEM((_C, _EMB), jnp.float32),
        pltpu.SemaphoreType.DMA,
        pltpu.SemaphoreType.DMA,
    ],
)(_sc_img_body)


def _mm_body(x_ref, w_ref, g_ref, o_ref):
    o_ref[...] = (
        jnp.dot(x_ref[...], w_ref[...], preferred_element_type=jnp.float32)
        + g_ref[...]
    )


def _img_project(x, w_t, gsum):
    m_blk = 1024
    return pl.pallas_call(
        _mm_body,
        grid=(_NI // m_blk,),
        in_specs=[
            pl.BlockSpec((m_blk, _IMG_IN), lambda i: (i, 0)),
            pl.BlockSpec((_IMG_IN, _EMB), lambda i: (0, 0)),
            pl.BlockSpec((m_blk, _EMB), lambda i: (i, 0)),
        ],
        out_specs=pl.BlockSpec((m_blk, _EMB), lambda i: (i, 0)),
        out_shape=jax.ShapeDtypeStruct((_NI, _EMB), jnp.float32),
    )(x, w_t, gsum)


def kernel(src_input, src_pos, src_turn, src_speaker, image_input,
           image_pos, image_turn, image_speaker, tok_table, pos_table, W_img):
    i32 = jnp.int32

    def lmajor(a, chunks):
        return a.T.astype(i32).reshape(_NW, chunks, _C)

    t0 = lmajor(src_input, _TCHUNKS)
    t1 = lmajor(src_turn, _TCHUNKS)
    t2 = lmajor(src_speaker, _TCHUNKS)
    tp = lmajor(src_pos, _TCHUNKS)
    i0 = lmajor(image_turn, _ICHUNKS)
    i1 = lmajor(image_speaker, _ICHUNKS)
    i2 = lmajor(image_pos, _ICHUNKS)

    img_gather = _sc_img(tok_table, i0, i1, i2)
    text_flat = _sc_text(tok_table, pos_table, t0, t1, t2, tp)

    x = image_input.transpose(1, 0, 2).reshape(_NI, _IMG_IN)
    img_flat = _img_project(x, W_img.T, img_gather)

    return (text_flat.reshape(_BT, _LT, _EMB),
            img_flat.reshape(_LI, _BT, _EMB).transpose(1, 0, 2))


# speedup vs baseline: 12.3137x; 1.1319x over previous
import functools

import jax
import jax.numpy as jnp
from jax import lax
from jax.experimental import pallas as pl
from jax.experimental.pallas import tpu as pltpu
from jax.experimental.pallas import tpu_sc as plsc

_EMB = 128
_NC, _NS = 2, 16
_NW = _NC * _NS
_C = 128

_BT, _LT = 1024, 200
_LI = 20
_NT = _BT * _LT
_NI = _BT * _LI
_TCHUNKS = _NT // (_NW * _C)
_ICHUNKS = _NI // (_NW * _C)
_IMG_IN = 2048

_MESH = plsc.VectorSubcoreMesh(core_axis_name="c", subcore_axis_name="s")
_PARAMS = pltpu.CompilerParams(use_tc_tiling_on_sc=True)


def _sc_text_body(tok, post, t0, t1, t2, tp, text_out,
                  v_t0, v_t1, v_t2, v_tp, didx, pv,
                  bA, bB, bC, gsA, gsB, gsC, ssA, ssB, ssC):
    wid = lax.axis_index("s") * _NC + lax.axis_index("c")

    pltpu.sync_copy(t0.at[wid], v_t0)
    pltpu.sync_copy(t1.at[wid], v_t1)
    pltpu.sync_copy(t2.at[wid], v_t2)
    pltpu.sync_copy(tp.at[wid], v_tp)

    @pl.when(lax.axis_index("s") == 0)
    def _():
        pltpu.sync_copy(post, pv)

    plsc.subcore_barrier()

    lane = jnp.arange(16, dtype=jnp.int32) * _LT

    def dst_fill(c, k):
        gg = wid * _TCHUNKS + c
        base = (gg % (_BT // _C)) * _C * _LT + gg // (_BT // _C)
        for g in range(_C // 16):
            didx[k, pl.ds(g * 16, 16)] = lane + (base + g * 16 * _LT)

    def adds3(c, buf, gsem):
        pltpu.async_copy(tok.at[v_t0.at[c]], buf, gsem, add=True)
        pltpu.async_copy(tok.at[v_t1.at[c]], buf, gsem, add=True)
        pltpu.async_copy(tok.at[v_t2.at[c]], buf, gsem, add=True)

    def wait_adds3(c, buf, gsem):
        pltpu.make_async_copy(tok.at[v_t0.at[c]], buf, gsem).wait()
        pltpu.make_async_copy(tok.at[v_t1.at[c]], buf, gsem).wait()
        pltpu.make_async_copy(tok.at[v_t2.at[c]], buf, gsem).wait()

    def pos_fill(c, buf, gsem):
        pltpu.async_copy(pv.at[v_tp.at[c]], buf, gsem).wait()

    def scat(k, buf, ssem):
        return pltpu.async_copy(buf, text_out.at[didx.at[k]], ssem)

    def wait_scat(k, buf, ssem):
        pltpu.make_async_copy(buf, text_out.at[didx.at[k]], ssem).wait()

    bufs = (bA, bB, bC)
    gsems = (gsA, gsB, gsC)
    ssems = (ssA, ssB, ssC)
    n_slots = _TCHUNKS + 1
    n_iters = (n_slots + 2) // 3

    def ring(i, _):
        for k in range(3):
            c = 3 * i + k
            buf, gsem, ssem = bufs[k], gsems[k], ssems[k]

            @pl.when(jnp.logical_and(c >= 3, c < _TCHUNKS))
            def _():
                wait_scat(k, buf, ssem)

            @pl.when(c < _TCHUNKS)
            def _():
                dst_fill(c, k)
                pos_fill(c, buf, gsem)
                adds3(c, buf, gsem)

            p = c - 1
            kp = (k + 2) % 3
            pbuf, pgsem, pssem = bufs[kp], gsems[kp], ssems[kp]

            @pl.when(jnp.logical_and(p >= 0, p < _TCHUNKS))
            def _():
                wait_adds3(p, pbuf, pgsem)
                scat(kp, pbuf, pssem)

        return 0

    lax.fori_loop(0, n_iters, ring, 0)
    for c in (_TCHUNKS - 3, _TCHUNKS - 2, _TCHUNKS - 1):
        wait_scat(c % 3, bufs[c % 3], ssems[c % 3])


_sc_text = functools.partial(
    pl.kernel,
    out_type=jax.ShapeDtypeStruct((_NT, _EMB), jnp.float32),
    mesh=_MESH,
    compiler_params=_PARAMS,
    scratch_types=[
        pltpu.VMEM((_TCHUNKS, _C), jnp.int32),
        pltpu.VMEM((_TCHUNKS, _C), jnp.int32),
        pltpu.VMEM((_TCHUNKS, _C), jnp.int32),
        pltpu.VMEM((_TCHUNKS, _C), jnp.int32),
        pltpu.VMEM((3, _C), jnp.int32),
        pltpu.VMEM_SHARED((_LT + 1, _EMB), jnp.float32),
        pltpu.VMEM((_C, _EMB), jnp.float32),
        pltpu.VMEM((_C, _EMB), jnp.float32),
        pltpu.VMEM((_C, _EMB), jnp.float32),
        pltpu.SemaphoreType.DMA,
        pltpu.SemaphoreType.DMA,
        pltpu.SemaphoreType.DMA,
        pltpu.SemaphoreType.DMA,
        pltpu.SemaphoreType.DMA,
        pltpu.SemaphoreType.DMA,
    ],
)(_sc_text_body)


def _sc_img_body(tok, i0, i1, i2, img_out,
                 v_i0, v_i1, v_i2, bA, bB, gsA, gsB):
    wid = lax.axis_index("s") * _NC + lax.axis_index("c")

    pltpu.sync_copy(i0.at[wid], v_i0)
    pltpu.sync_copy(i1.at[wid], v_i1)
    pltpu.sync_copy(i2.at[wid], v_i2)

    def g0(c, buf, gsem):
        return pltpu.async_copy(tok.at[v_i0.at[c]], buf, gsem)

    def adds(c, buf, gsem):
        pltpu.async_copy(tok.at[v_i1.at[c]], buf, gsem, add=True)
        pltpu.async_copy(tok.at[v_i2.at[c]], buf, gsem, add=True)

    def wait_adds(c, buf, gsem):
        pltpu.make_async_copy(tok.at[v_i1.at[c]], buf, gsem).wait()
        pltpu.make_async_copy(tok.at[v_i2.at[c]], buf, gsem).wait()

    def out_at(c):
        return img_out.at[pl.ds(wid * (_ICHUNKS * _C) + c * _C, _C)]

    bufs = (bA, bB)
    sems = (gsA, gsB)
    g0(0, bA, gsA).wait()
    adds(0, bA, gsA)
    for c in range(1, _ICHUNKS + 1):
        if c < _ICHUNKS:
            g0(c, bufs[c % 2], sems[c % 2]).wait()
            adds(c, bufs[c % 2], sems[c % 2])
        p = c - 1
        wait_adds(p, bufs[p % 2], sems[p % 2])
        pltpu.sync_copy(bufs[p % 2], out_at(p))


_sc_img = functools.partial(
    pl.kernel,
    out_type=jax.ShapeDtypeStruct((_NI, _EMB), jnp.float32),
    mesh=_MESH,
    compiler_params=_PARAMS,
    scratch_types=[
        pltpu.VMEM((_ICHUNKS, _C), jnp.int32),
        pltpu.VMEM((_ICHUNKS, _C), jnp.int32),
        pltpu.VMEM((_ICHUNKS, _C), jnp.int32),
        pltpu.VMEM((_C, _EMB), jnp.float32),
        pltpu.VMEM((_C, _EMB), jnp.float32),
        pltpu.SemaphoreType.DMA,
        pltpu.SemaphoreType.DMA,
    ],
)(_sc_img_body)


def _mm_body(x_ref, w_ref, g_ref, o_ref):
    o_ref[...] = (
        jnp.dot(x_ref[...], w_ref[...], preferred_element_type=jnp.float32)
        + g_ref[...]
    )


def _img_project(x, w_t, gsum):
    m_blk = 1024
    return pl.pallas_call(
        _mm_body,
        grid=(_NI // m_blk,),
        in_specs=[
            pl.BlockSpec((m_blk, _IMG_IN), lambda i: (i, 0)),
            pl.BlockSpec((_IMG_IN, _EMB), lambda i: (0, 0)),
            pl.BlockSpec((m_blk, _EMB), lambda i: (i, 0)),
        ],
        out_specs=pl.BlockSpec((m_blk, _EMB), lambda i: (i, 0)),
        out_shape=jax.ShapeDtypeStruct((_NI, _EMB), jnp.float32),
    )(x, w_t, gsum)


def kernel(src_input, src_pos, src_turn, src_speaker, image_input,
           image_pos, image_turn, image_speaker, tok_table, pos_table, W_img):
    i32 = jnp.int32

    def lmajor(a, chunks):
        return a.T.astype(i32).reshape(_NW, chunks, _C)

    t0 = lmajor(src_input, _TCHUNKS)
    t1 = lmajor(src_turn, _TCHUNKS)
    t2 = lmajor(src_speaker, _TCHUNKS)
    tp = lmajor(src_pos, _TCHUNKS)
    i0 = lmajor(image_turn, _ICHUNKS)
    i1 = lmajor(image_speaker, _ICHUNKS)
    i2 = lmajor(image_pos, _ICHUNKS)

    img_gather = _sc_img(tok_table, i0, i1, i2)
    text_flat = _sc_text(tok_table, pos_table, t0, t1, t2, tp)

    x = image_input.transpose(1, 0, 2).reshape(_NI, _IMG_IN)
    img_flat = _img_project(x, W_img.T, img_gather)

    return (text_flat.reshape(_BT, _LT, _EMB),
            img_flat.reshape(_LI, _BT, _EMB).transpose(1, 0, 2))
